# baseline (speedup 1.0000x reference)
import functools

import jax
import jax.numpy as jnp
from jax import lax
from jax.experimental import pallas as pl
from jax.experimental.pallas import tpu as pltpu
from jax.experimental.pallas import tpu_sc as plsc

_BATCH = 4096
_SEQ_LEN = 200
_EMSIZE = 64

_NC = 2
_NS = 16
_NW = _NC * _NS

_B_TOTAL = _BATCH * _SEQ_LEN
_CHUNK = 128
_B_PER_W = _B_TOTAL // _NW
_N_CHUNKS = _B_PER_W // _CHUNK
_K = 2
_SLOTS = 4
_NGROUP = _N_CHUNKS // _K

_mesh = plsc.VectorSubcoreMesh(core_axis_name="c", subcore_axis_name="s")


@functools.partial(
    pl.kernel,
    out_type=jax.ShapeDtypeStruct((_B_TOTAL, _EMSIZE), jnp.float32),
    mesh=_mesh,
    scratch_types=[
        pltpu.VMEM((_N_CHUNKS, _CHUNK), jnp.int32),
        pltpu.VMEM((_SLOTS, _K, _CHUNK, _EMSIZE), jnp.float32),
        pltpu.SemaphoreType.DMA((_SLOTS,)),
        pltpu.SemaphoreType.DMA((_SLOTS,)),
    ],
    compiler_params=pltpu.CompilerParams(use_tc_tiling_on_sc=False),
)
def _embed_sc(idx_hbm, table_hbm, out_hbm, idx_v, rows_v, gsem, wsem):
    wid = lax.axis_index("s") * _NC + lax.axis_index("c")
    chunk0 = wid * _N_CHUNKS
    pltpu.sync_copy(idx_hbm.at[pl.ds(chunk0, _N_CHUNKS), :], idx_v)

    def _issue_gathers(g, s):
        for k in range(_K):
            pltpu.async_copy(
                table_hbm.at[idx_v.at[g * _K + k]], rows_v.at[s, k], gsem.at[s]
            )

    def _drain_gathers(g, s):
        for k in range(_K):
            pltpu.make_async_copy(
                table_hbm.at[idx_v.at[g * _K + k]], rows_v.at[s, k], gsem.at[s]
            ).wait()

    def _issue_writes(g, s):
        for k in range(_K):
            row0 = (chunk0 + g * _K + k) * _CHUNK
            pltpu.async_copy(
                rows_v.at[s, k], out_hbm.at[pl.ds(row0, _CHUNK), :], wsem.at[s]
            )

    def _drain_writes(g, s):
        for k in range(_K):
            row0 = (chunk0 + g * _K + k) * _CHUNK
            pltpu.make_async_copy(
                rows_v.at[s, k], out_hbm.at[pl.ds(row0, _CHUNK), :], wsem.at[s]
            ).wait()

    def _body(g, s, drain_old, issue_ahead):
        _drain_gathers(g, s)
        _issue_writes(g, s)
        if drain_old:
            _drain_writes(g - 2, (s + 2) % _SLOTS)
        if issue_ahead:
            _issue_gathers(g + 2, (s + 2) % _SLOTS)

    _issue_gathers(0, 0)
    _issue_gathers(1, 1)
    _body(0, 0, False, True)
    _body(1, 1, False, True)

    @pl.loop(0, (_NGROUP - 4) // _SLOTS)
    def _steady(t):
        g0 = 2 + t * _SLOTS
        for q in range(_SLOTS):
            _body(g0 + q, (2 + q) % _SLOTS, True, True)

    _body(_NGROUP - 2, (_NGROUP - 2) % _SLOTS, True, False)
    _body(_NGROUP - 1, (_NGROUP - 1) % _SLOTS, True, False)
    _drain_writes(_NGROUP - 2, (_NGROUP - 2) % _SLOTS)
    _drain_writes(_NGROUP - 1, (_NGROUP - 1) % _SLOTS)


def kernel(sequence, src_word_table):
    idx = sequence.reshape(_B_TOTAL // _CHUNK, _CHUNK)
    out = _embed_sc(idx, src_word_table)
    return out.reshape(_BATCH, _SEQ_LEN, _EMSIZE)
